# baseline (speedup 1.0000x reference)
import functools

import jax
import jax.numpy as jnp
from jax import lax
from jax.experimental import pallas as pl
from jax.experimental.pallas import tpu as pltpu
from jax.experimental.pallas import tpu_sc as plsc

B, E, N, D = 16, 32000, 1000, 64
NP = 1024
NH = NP // 2
L = 16
CH = 3200
NCH = E // CH
MCAP = 2048

_sc_mesh = plsc.VectorSubcoreMesh(core_axis_name="c", subcore_axis_name="s")


def _sc_body(ei_hbm, act_hbm, ea_hbm, gath_out, win_out,
             rc_v, win_v, wsp_v, mev_v, mcol_v, tbuf, rows_v, act_v, sem):
    c = lax.axis_index("c")
    s = lax.axis_index("s")

    @pl.when(s < 8)
    def _():
        b = c * 8 + s
        lanes = lax.iota(jnp.int32, L)
        pltpu.sync_copy(act_hbm, act_v)
        a = plsc.load_gather(act_v, [jnp.full((L,), b, jnp.int32)])

        def init_body(i, _):
            win_v[pl.ds(i * L, L)] = jnp.full((L,), -1, jnp.int32)
            return 0
        lax.fori_loop(0, NP // L, init_body, 0)

        cnt = jnp.int32(0)
        for g in range(NCH):
            pltpu.sync_copy(ei_hbm.at[b, :, pl.ds(g * CH, CH)], rc_v)

            def scan_body(i, cnt, g=g):
                r = rc_v[0, pl.ds(i * L, L)]
                m = r == a
                pc = plsc.all_reduce_population_count(m)[0]

                @pl.when(pc > 0)
                def _():
                    cidx = rc_v[1, pl.ds(i * L, L)]
                    ev = jnp.int32(g * CH) + i * L + lanes
                    cl = jnp.minimum(cnt, MCAP)
                    plsc.store_scatter(win_v, [cidx], ev, mask=m)
                    plsc.store_compressed(mev_v.at[pl.ds(cl, L)], ev, mask=m)
                    plsc.store_compressed(mcol_v.at[pl.ds(cl, L)], cidx,
                                          mask=m)
                return cnt + pc
            cnt = lax.fori_loop(0, CH // L, scan_body, cnt)

        def fetch_body(i, _):
            w = mev_v[pl.ds(i, L)][0]
            cc = mcol_v[pl.ds(i, L)][0]
            w128 = pl.multiple_of((w // 128) * 128, 128)
            pltpu.sync_copy(ea_hbm.at[b, :, pl.ds(w128, 128)], tbuf)
            wsub = jnp.full((L,), w - w128, jnp.int32)
            half = (cc % 2) * D
            for k in range(D // L):
                rows_v[cc // 2, pl.ds(half + k * L, L)] = \
                    plsc.load_gather(tbuf, [lanes + k * L, wsub])
            return 0
        lax.fori_loop(0, jnp.minimum(cnt, MCAP), fetch_body, 0)

        for j in range(NH // L):
            idx2 = (j * L + lanes) * 2
            wsp_v[pl.ds(j * L, L)] = plsc.load_gather(win_v, [idx2])
            wsp_v[pl.ds(NH + j * L, L)] = plsc.load_gather(win_v, [idx2 + 1])

        pltpu.sync_copy(rows_v, gath_out.at[pl.ds(b * NH, NH)])
        pltpu.sync_copy(wsp_v.at[pl.ds(0, NH)],
                        win_out.at[pl.ds(b * NH, NH)])
        pltpu.sync_copy(wsp_v.at[pl.ds(NH, NH)],
                        win_out.at[pl.ds(B * NH + b * NH, NH)])


_sc_select = pl.kernel(
    _sc_body,
    out_type=(
        jax.ShapeDtypeStruct((B * NH, 2 * D), jnp.float32),
        jax.ShapeDtypeStruct((2 * B * NH,), jnp.int32),
    ),
    mesh=_sc_mesh,
    compiler_params=pltpu.CompilerParams(needs_layout_passes=False),
    scratch_types=[
        pltpu.VMEM((2, CH), jnp.int32),
        pltpu.VMEM((NP,), jnp.int32),
        pltpu.VMEM((NP,), jnp.int32),
        pltpu.VMEM((MCAP + L,), jnp.int32),
        pltpu.VMEM((MCAP + L,), jnp.int32),
        pltpu.VMEM((D, 128), jnp.float32),
        pltpu.VMEM((NH, 2 * D), jnp.float32),
        pltpu.VMEM((L,), jnp.int32),
        pltpu.SemaphoreType.DMA,
    ],
)


def _mlp_body(g_ref, wn_ref, am_ref, w0_ref, b0_ref, w1_ref, b1_ref,
              w2_ref, b2_ref, wo_ref, bo_ref, lp_ref, mk_ref):
    x = g_ref[...]
    for w_r, b_r in ((w0_ref, b0_ref), (w1_ref, b1_ref), (w2_ref, b2_ref)):
        y = lax.dot_general(x, w_r[...], (((1,), (0,)), ((), ())),
                            preferred_element_type=jnp.float32)
        y = y + b_r[...][None, :]
        x = y * jax.nn.sigmoid(y)
    prod = x * wo_ref[...][None, :]
    bo = bo_ref[0]
    wn = wn_ref[...]
    am = am_ref[...]
    l_even = jnp.sum(prod[:, :D], axis=1) + bo
    l_odd = jnp.sum(prod[:, D:], axis=1) + bo
    logits = jnp.concatenate([l_even, l_odd])
    lp = jnp.where(wn >= 0, logits, jnp.float32(-1e9))
    lp_ref[...] = jnp.where(am == 0, jnp.float32(-jnp.inf), lp)
    mk_ref[...] = (am == 0).astype(jnp.int8)


_mlp_call = pl.pallas_call(
    _mlp_body,
    out_shape=[
        jax.ShapeDtypeStruct((B * NP,), jnp.float32),
        jax.ShapeDtypeStruct((B * NP,), jnp.int8),
    ],
)


def kernel(edge_attr, edge_index, action, action_mask,
           W0, b0, W1, b1, W2, b2, Wout, bout):
    act = action.astype(jnp.int32)
    gath, win = _sc_select(edge_index.astype(jnp.int32), act,
                           edge_attr.transpose(0, 2, 1))
    am_pad = jnp.pad(action_mask, ((0, 0), (0, NP - N))).astype(jnp.int32)
    am2 = am_pad.reshape(B * NH, 2)
    am_cat = jnp.concatenate([am2[:, 0], am2[:, 1]])
    z = jnp.zeros((D, D), jnp.float32)
    wps = [jnp.block([[W.T, z], [z, W.T]]) for W in (W0, W1, W2)]
    bps = [jnp.concatenate([bb, bb]) for bb in (b0, b1, b2)]
    wo2 = jnp.concatenate([Wout[0], Wout[0]])
    lp_flat, mk_flat = _mlp_call(
        gath, win, am_cat, wps[0], bps[0], wps[1], bps[1], wps[2], bps[2],
        wo2, bout)
    lp2 = jnp.stack([lp_flat[:B * NH].reshape(B, NH),
                     lp_flat[B * NH:].reshape(B, NH)], axis=-1)
    mk2 = jnp.stack([mk_flat[:B * NH].reshape(B, NH),
                     mk_flat[B * NH:].reshape(B, NH)], axis=-1)
    log_p = lp2.reshape(B, NP)[:, :N]
    mask = mk2.reshape(B, NP)[:, :N].astype(bool)
    return log_p, mask


# speedup vs baseline: 1.5075x; 1.5075x over previous
import functools

import jax
import jax.numpy as jnp
from jax import lax
from jax.experimental import pallas as pl
from jax.experimental.pallas import tpu as pltpu
from jax.experimental.pallas import tpu_sc as plsc

B, E, N, D = 16, 32000, 1000, 64
E2 = E // 2
NP = 1024
NH = NP // 2
L = 16
CH = 3200
NCH = E2 // CH
MCAP = 2048
GW = B * NH

_sc_mesh = plsc.VectorSubcoreMesh(core_axis_name="c", subcore_axis_name="s")


def _sc_body(ei_hbm, act_hbm, ea_hbm, gath_out, win_out,
             rc_v, win_v, wsp_v, mev_v, mcol_v, tbuf, rows_v, act_v, sem):
    h = lax.axis_index("c")
    b = lax.axis_index("s")
    lanes = lax.iota(jnp.int32, L)
    pltpu.sync_copy(act_hbm, act_v)
    a = plsc.load_gather(act_v, [jnp.full((L,), b, jnp.int32)])
    e0 = h * E2

    def init_body(i, _):
        win_v[pl.ds(i * L, L)] = jnp.full((L,), -1, jnp.int32)
        return 0
    lax.fori_loop(0, NP // L, init_body, 0)

    cnt = jnp.int32(0)
    for g in range(NCH):
        pltpu.sync_copy(ei_hbm.at[b, :, pl.ds(e0 + g * CH, CH)], rc_v)

        def scan_body(i, cnt, g=g):
            r = rc_v[0, pl.ds(i * L, L)]
            cidx = rc_v[1, pl.ds(i * L, L)]
            ev = e0 + jnp.int32(g * CH) + i * L + lanes
            m = r == a
            cl = jnp.minimum(cnt, MCAP)
            plsc.store_scatter(win_v, [cidx], ev, mask=m)
            plsc.store_compressed(mev_v.at[pl.ds(cl, L)], ev, mask=m)
            plsc.store_compressed(mcol_v.at[pl.ds(cl, L)], cidx, mask=m)
            return cnt + plsc.all_reduce_population_count(m)[0]
        cnt = lax.fori_loop(0, CH // L, scan_body, cnt)

    def fetch_body(i, _):
        w = mev_v[pl.ds(i, L)][0]
        cc = mcol_v[pl.ds(i, L)][0]
        w128 = pl.multiple_of((w // 128) * 128, 128)
        pltpu.sync_copy(ea_hbm.at[b, :, pl.ds(w128, 128)], tbuf)
        wsub = jnp.full((L,), w - w128, jnp.int32)
        half = (cc % 2) * D
        for k in range(D // L):
            rows_v[cc // 2, pl.ds(half + k * L, L)] = \
                plsc.load_gather(tbuf, [lanes + k * L, wsub])
        return 0
    lax.fori_loop(0, jnp.minimum(cnt, MCAP), fetch_body, 0)

    for j in range(NH // L):
        idx2 = (j * L + lanes) * 2
        wsp_v[pl.ds(j * L, L)] = plsc.load_gather(win_v, [idx2])
        wsp_v[pl.ds(NH + j * L, L)] = plsc.load_gather(win_v, [idx2 + 1])

    pltpu.sync_copy(rows_v, gath_out.at[pl.ds(h * GW + b * NH, NH)])
    pltpu.sync_copy(wsp_v.at[pl.ds(0, NH)],
                    win_out.at[pl.ds(h * 2 * GW + b * NH, NH)])
    pltpu.sync_copy(wsp_v.at[pl.ds(NH, NH)],
                    win_out.at[pl.ds(h * 2 * GW + GW + b * NH, NH)])


_sc_select = pl.kernel(
    _sc_body,
    out_type=(
        jax.ShapeDtypeStruct((2 * GW, 2 * D), jnp.float32),
        jax.ShapeDtypeStruct((4 * GW,), jnp.int32),
    ),
    mesh=_sc_mesh,
    compiler_params=pltpu.CompilerParams(needs_layout_passes=False),
    scratch_types=[
        pltpu.VMEM((2, CH), jnp.int32),
        pltpu.VMEM((NP,), jnp.int32),
        pltpu.VMEM((NP,), jnp.int32),
        pltpu.VMEM((MCAP + L,), jnp.int32),
        pltpu.VMEM((MCAP + L,), jnp.int32),
        pltpu.VMEM((D, 128), jnp.float32),
        pltpu.VMEM((NH, 2 * D), jnp.float32),
        pltpu.VMEM((L,), jnp.int32),
        pltpu.SemaphoreType.DMA,
    ],
)


def _mlp_body(g_ref, wn_ref, am_ref, w0_ref, b0_ref, w1_ref, b1_ref,
              w2_ref, b2_ref, wo_ref, bo_ref, lp_ref, mk_ref):
    g = g_ref[...]
    wn = wn_ref[...]
    am = am_ref[...]
    w0h = wn[:2 * GW]
    w1h = wn[2 * GW:]
    g0 = g[:GW]
    g1 = g[GW:]
    wvec = wo_ref[...][0]
    bo = bo_ref[0]
    outs = []
    for x in (g0[:, :D], g0[:, D:], g1[:, :D], g1[:, D:]):
        for w_r, b_r in ((w0_ref, b0_ref), (w1_ref, b1_ref),
                         (w2_ref, b2_ref)):
            y = lax.dot_general(x, w_r[...], (((1,), (1,)), ((), ())),
                                preferred_element_type=jnp.float32)
            y = y + b_r[...][None, :]
            x = y * jax.nn.sigmoid(y)
        outs.append(jnp.sum(x * wvec[None, :], axis=1) + bo)
    l0 = jnp.concatenate(outs[:2])
    l1 = jnp.concatenate(outs[2:])
    logits = jnp.where(w1h >= 0, l1, l0)
    wm = jnp.maximum(w0h, w1h)
    lp = jnp.where(wm >= 0, logits, jnp.float32(-1e9))
    lp_ref[...] = jnp.where(am == 0, jnp.float32(-jnp.inf), lp)
    mk_ref[...] = (am == 0).astype(jnp.int8)


_mlp_call = pl.pallas_call(
    _mlp_body,
    out_shape=[
        jax.ShapeDtypeStruct((B * NP,), jnp.float32),
        jax.ShapeDtypeStruct((B * NP,), jnp.int8),
    ],
)


def kernel(edge_attr, edge_index, action, action_mask,
           W0, b0, W1, b1, W2, b2, Wout, bout):
    act = action.astype(jnp.int32)
    gath, win = _sc_select(edge_index.astype(jnp.int32), act,
                           edge_attr.transpose(0, 2, 1))
    am_pad = jnp.pad(action_mask, ((0, 0), (0, NP - N))).astype(jnp.int32)
    am2 = am_pad.reshape(GW, 2)
    am_cat = jnp.concatenate([am2[:, 0], am2[:, 1]])
    lp_flat, mk_flat = _mlp_call(
        gath, win, am_cat, W0, b0, W1, b1, W2, b2, Wout, bout)
    lp2 = jnp.stack([lp_flat[:GW].reshape(B, NH),
                     lp_flat[GW:].reshape(B, NH)], axis=-1)
    mk2 = jnp.stack([mk_flat[:GW].reshape(B, NH),
                     mk_flat[GW:].reshape(B, NH)], axis=-1)
    log_p = lp2.reshape(B, NP)[:, :N]
    mask = mk2.reshape(B, NP)[:, :N].astype(bool)
    return log_p, mask


# speedup vs baseline: 1.6340x; 1.0839x over previous
import functools

import jax
import jax.numpy as jnp
from jax import lax
from jax.experimental import pallas as pl
from jax.experimental.pallas import tpu as pltpu
from jax.experimental.pallas import tpu_sc as plsc

B, E, N, D = 16, 32000, 1000, 64
E2 = E // 2
NP = 1024
NH = NP // 2
L = 16
CH = 3200
NCH = E2 // CH
MCAP = 2048
GW = B * NH

_sc_mesh = plsc.VectorSubcoreMesh(core_axis_name="c", subcore_axis_name="s")


def _sc_body(ei_hbm, act_hbm, ea_hbm, gath_out, win_out,
             rc_v, win_v, wsp_v, mev_v, mcol_v, tbuf, rows_v, act_v,
             nwin_v, nmc_v, ncv_v, grp_v,
             sh_rows, sh_win, sh_mc, sh_cnt, sem):
    cc_ax = lax.axis_index("c")
    s = lax.axis_index("s")
    b = cc_ax * 8 + s // 2
    h = s % 2
    lanes = lax.iota(jnp.int32, L)
    pltpu.sync_copy(act_hbm, act_v)
    a = plsc.load_gather(act_v, [jnp.full((L,), b, jnp.int32)])
    e0 = h * E2

    def init_body(i, _):
        win_v[pl.ds(i * L, L)] = jnp.full((L,), -1, jnp.int32)
        return 0
    lax.fori_loop(0, NP // L, init_body, 0)

    cnt = jnp.int32(0)
    for g in range(NCH):
        pltpu.sync_copy(ei_hbm.at[b, :, pl.ds(e0 + g * CH, CH)], rc_v)

        def scan_body(i, cnt, g=g):
            r = rc_v[0, pl.ds(i * L, L)]
            cidx = rc_v[1, pl.ds(i * L, L)]
            ev = e0 + jnp.int32(g * CH) + i * L + lanes
            m = r == a
            cl = jnp.minimum(cnt, MCAP)
            plsc.store_scatter(win_v, [cidx], ev, mask=m)
            plsc.store_compressed(mev_v.at[pl.ds(cl, L)], ev, mask=m)
            plsc.store_compressed(mcol_v.at[pl.ds(cl, L)], cidx, mask=m)
            return cnt + plsc.all_reduce_population_count(m)[0]
        cnt = lax.fori_loop(0, CH // L, scan_body, cnt)
    cnt = jnp.minimum(cnt, MCAP)

    def fetch_body(i, _):
        w = mev_v[pl.ds(i, L)][0]
        cc = mcol_v[pl.ds(i, L)][0]
        w128 = pl.multiple_of((w // 128) * 128, 128)
        pltpu.sync_copy(ea_hbm.at[b, :, pl.ds(w128, 128)], tbuf)
        wsub = jnp.full((L,), w - w128, jnp.int32)
        half = (cc % 2) * D
        for k in range(D // L):
            rows_v[cc // 2, pl.ds(half + k * L, L)] = \
                plsc.load_gather(tbuf, [lanes + k * L, wsub])
        return 0
    lax.fori_loop(0, cnt, fetch_body, 0)

    p = s // 2

    @pl.when(h == 1)
    def _():
        ncv_v[...] = jnp.full((L,), cnt, jnp.int32)
        pltpu.sync_copy(rows_v, sh_rows.at[pl.ds(p * NH, NH)])
        pltpu.sync_copy(win_v, sh_win.at[pl.ds(p * NP, NP)])
        pltpu.sync_copy(mcol_v, sh_mc.at[pl.ds(p * (MCAP + L), MCAP + L)])
        pltpu.sync_copy(ncv_v, sh_cnt.at[pl.ds(p * L, L)])

    plsc.subcore_barrier()

    @pl.when(h == 0)
    def _():
        pltpu.sync_copy(sh_win.at[pl.ds(p * NP, NP)], nwin_v)
        pltpu.sync_copy(sh_mc.at[pl.ds(p * (MCAP + L), MCAP + L)], nmc_v)
        pltpu.sync_copy(sh_cnt.at[pl.ds(p * L, L)], ncv_v)
        ncnt = jnp.minimum(ncv_v[...][0], MCAP)

        def merge_body(i, _):
            mc = nmc_v[pl.ds(i, L)][0]
            pr = mc // 2
            g8 = pl.multiple_of((pr // 8) * 8, 8)
            pltpu.sync_copy(sh_rows.at[pl.ds(p * NH + g8, 8)], grp_v)
            half = (mc % 2) * D
            for k in range(D // L):
                rows_v[pr, pl.ds(half + k * L, L)] = \
                    grp_v[pr - g8, pl.ds(half + k * L, L)]
            return 0
        lax.fori_loop(0, ncnt, merge_body, 0)

        def wmerge_body(i, _):
            win_v[pl.ds(i * L, L)] = jnp.maximum(
                win_v[pl.ds(i * L, L)], nwin_v[pl.ds(i * L, L)])
            return 0
        lax.fori_loop(0, NP // L, wmerge_body, 0)

        for j in range(NH // L):
            idx2 = (j * L + lanes) * 2
            wsp_v[pl.ds(j * L, L)] = plsc.load_gather(win_v, [idx2])
            wsp_v[pl.ds(NH + j * L, L)] = \
                plsc.load_gather(win_v, [idx2 + 1])

        pltpu.sync_copy(rows_v, gath_out.at[pl.ds(b * NH, NH)])
        pltpu.sync_copy(wsp_v.at[pl.ds(0, NH)],
                        win_out.at[pl.ds(b * NH, NH)])
        pltpu.sync_copy(wsp_v.at[pl.ds(NH, NH)],
                        win_out.at[pl.ds(GW + b * NH, NH)])


_sc_select = pl.kernel(
    _sc_body,
    out_type=(
        jax.ShapeDtypeStruct((GW, 2 * D), jnp.float32),
        jax.ShapeDtypeStruct((2 * GW,), jnp.int32),
    ),
    mesh=_sc_mesh,
    compiler_params=pltpu.CompilerParams(needs_layout_passes=False),
    scratch_types=[
        pltpu.VMEM((2, CH), jnp.int32),
        pltpu.VMEM((NP,), jnp.int32),
        pltpu.VMEM((NP,), jnp.int32),
        pltpu.VMEM((MCAP + L,), jnp.int32),
        pltpu.VMEM((MCAP + L,), jnp.int32),
        pltpu.VMEM((D, 128), jnp.float32),
        pltpu.VMEM((NH, 2 * D), jnp.float32),
        pltpu.VMEM((L,), jnp.int32),
        pltpu.VMEM((NP,), jnp.int32),
        pltpu.VMEM((MCAP + L,), jnp.int32),
        pltpu.VMEM((L,), jnp.int32),
        pltpu.VMEM((8, 2 * D), jnp.float32),
        pltpu.VMEM_SHARED((8 * NH, 2 * D), jnp.float32),
        pltpu.VMEM_SHARED((8 * NP,), jnp.int32),
        pltpu.VMEM_SHARED((8 * (MCAP + L),), jnp.int32),
        pltpu.VMEM_SHARED((8 * L,), jnp.int32),
        pltpu.SemaphoreType.DMA,
    ],
)


def _mlp_body(g_ref, wn_ref, am_ref, w0_ref, b0_ref, w1_ref, b1_ref,
              w2_ref, b2_ref, wo_ref, bo_ref, lp_ref, mk_ref):
    g = g_ref[...]
    wn = wn_ref[...]
    am = am_ref[...]
    wvec = wo_ref[...][0]
    bo = bo_ref[0]
    outs = []
    for x in (g[:, :D], g[:, D:]):
        for w_r, b_r in ((w0_ref, b0_ref), (w1_ref, b1_ref),
                         (w2_ref, b2_ref)):
            y = lax.dot_general(x, w_r[...], (((1,), (1,)), ((), ())),
                                preferred_element_type=jnp.float32)
            y = y + b_r[...][None, :]
            x = y * jax.nn.sigmoid(y)
        outs.append(jnp.sum(x * wvec[None, :], axis=1) + bo)
    logits = jnp.concatenate(outs)
    lp = jnp.where(wn >= 0, logits, jnp.float32(-1e9))
    lp_ref[...] = jnp.where(am == 0, jnp.float32(-jnp.inf), lp)
    mk_ref[...] = (am == 0).astype(jnp.int8)


_mlp_call = pl.pallas_call(
    _mlp_body,
    out_shape=[
        jax.ShapeDtypeStruct((B * NP,), jnp.float32),
        jax.ShapeDtypeStruct((B * NP,), jnp.int8),
    ],
)


def kernel(edge_attr, edge_index, action, action_mask,
           W0, b0, W1, b1, W2, b2, Wout, bout):
    act = action.astype(jnp.int32)
    gath, win = _sc_select(edge_index.astype(jnp.int32), act,
                           edge_attr.transpose(0, 2, 1))
    am_pad = jnp.pad(action_mask, ((0, 0), (0, NP - N))).astype(jnp.int32)
    am2 = am_pad.reshape(GW, 2)
    am_cat = jnp.concatenate([am2[:, 0], am2[:, 1]])
    lp_flat, mk_flat = _mlp_call(
        gath, win, am_cat, W0, b0, W1, b1, W2, b2, Wout, bout)
    lp2 = jnp.stack([lp_flat[:GW].reshape(B, NH),
                     lp_flat[GW:].reshape(B, NH)], axis=-1)
    mk2 = jnp.stack([mk_flat[:GW].reshape(B, NH),
                     mk_flat[GW:].reshape(B, NH)], axis=-1)
    log_p = lp2.reshape(B, NP)[:, :N]
    mask = mk2.reshape(B, NP)[:, :N].astype(bool)
    return log_p, mask
